# Initial kernel scaffold; baseline (speedup 1.0000x reference)
import functools

import jax
import jax.numpy as jnp
from jax import lax
from jax.experimental import pallas as pl
from jax.experimental.pallas import tpu as pltpu
from jax.experimental.pallas import tpu_sc as plsc

NUM_EMB = 1_000_000
DIM = 128
NCB = 8
CBS = 256
SUB = 16
BATCH = 16384

_INFO = plsc.get_sparse_core_info()
NC, NS, L = _INFO.num_cores, _INFO.num_subcores, _INFO.num_lanes
NW = NC * NS
BPW = BATCH // NW
CHUNK = 128
NIC = BPW // CHUNK
NGC = BPW * NCB // CHUNK


def _pq_body(idx_hbm, cb_hbm, codes_hbm, out_hbm, idx_v, codes_v, g_v, rows_v, sem):
    wid = lax.axis_index("s") * NC + lax.axis_index("c")

    pltpu.sync_copy(idx_hbm.at[pl.ds(wid * NIC, NIC)], idx_v)

    handles = []
    for j in range(NIC):
        handles.append(
            pltpu.async_copy(codes_hbm.at[idx_v.at[j]],
                             codes_v.at[pl.ds(j * CHUNK, CHUNK)], sem))
    for h in handles:
        h.wait()

    lane = lax.iota(jnp.int32, L)
    col = lane & (NCB - 1)
    off = col * CBS

    def g_chunk(j, carry):
        jv = jnp.full((L,), j, dtype=jnp.int32)
        for l in range(CHUNK // L):
            row = j * (CHUNK // NCB) + l * (L // NCB) + (lane >> 3)
            c = plsc.load_gather(codes_v, [row, col])
            plsc.store_scatter(g_v, [jv, l * L + lane], c + off)
        return carry

    lax.fori_loop(0, NGC, g_chunk, 0)

    def fire(j, carry):
        pltpu.async_copy(cb_hbm.at[g_v.at[j]],
                         rows_v.at[pl.ds(j * CHUNK, CHUNK)], sem)
        return carry

    lax.fori_loop(0, NGC, fire, 0)

    def drain(j, carry):
        pltpu.make_async_copy(cb_hbm.at[g_v.at[j]],
                              rows_v.at[pl.ds(j * CHUNK, CHUNK)], sem).wait()
        return carry

    lax.fori_loop(0, NGC, drain, 0)

    pltpu.sync_copy(rows_v, out_hbm.at[pl.ds(wid * BPW * NCB, BPW * NCB)])


_pq_decode = functools.partial(
    pl.kernel,
    out_type=jax.ShapeDtypeStruct((BATCH * NCB, SUB), jnp.float32),
    mesh=plsc.VectorSubcoreMesh(core_axis_name="c", subcore_axis_name="s"),
    scratch_types=[
        pltpu.VMEM((NIC, CHUNK), jnp.int32),
        pltpu.VMEM((BPW, NCB), jnp.int32),
        pltpu.VMEM((NGC, CHUNK), jnp.int32),
        pltpu.VMEM((BPW * NCB, SUB), jnp.float32),
        pltpu.SemaphoreType.DMA,
    ],
)(_pq_body)


def kernel(indices, codebooks, codes):
    idx2 = indices.astype(jnp.int32).reshape(BATCH // CHUNK, CHUNK)
    cb_flat = codebooks.reshape(NCB * CBS, SUB)
    out = _pq_decode(idx2, cb_flat, codes)
    return out.reshape(BATCH, DIM)


# speedup vs baseline: 1.1694x; 1.1694x over previous
import functools

import jax
import jax.numpy as jnp
from jax import lax
from jax.experimental import pallas as pl
from jax.experimental.pallas import tpu as pltpu
from jax.experimental.pallas import tpu_sc as plsc

NUM_EMB = 1_000_000
DIM = 128
NCB = 8
CBS = 256
SUB = 16
BATCH = 16384

_INFO = plsc.get_sparse_core_info()
NC, NS, L = _INFO.num_cores, _INFO.num_subcores, _INFO.num_lanes
NW = NC * NS
BPW = BATCH // NW
CHUNK = 128
NIC = BPW // CHUNK
NGC = BPW * NCB // CHUNK


def _pq_body(idx_hbm, cb_hbm, codes_hbm, out_hbm, idx_v, codes_v, g_v,
             rows_v, sem):
    wid = lax.axis_index("s") * NC + lax.axis_index("c")

    pltpu.sync_copy(idx_hbm.at[pl.ds(wid * NIC, NIC)], idx_v)

    handles = []
    for j in range(NIC):
        handles.append(
            pltpu.async_copy(codes_hbm.at[idx_v.at[j]],
                             codes_v.at[pl.ds(j * CHUNK, CHUNK)], sem))
    for h in handles:
        h.wait()

    lane = lax.iota(jnp.int32, L)
    col = lane & (NCB - 1)
    off = col * CBS

    def g_chunk(j, carry):
        jv = jnp.full((L,), 0, dtype=jnp.int32) + j
        for l in range(CHUNK // L):
            row = j * (CHUNK // NCB) + l * (L // NCB) + (lane >> 3)
            c = plsc.load_gather(codes_v, [row, col])
            plsc.store_scatter(g_v, [jv, l * L + lane], c + off)
        return carry

    lax.fori_loop(0, NGC, g_chunk, 0)

    def fire(j, carry):
        pltpu.async_copy(cb_hbm.at[g_v.at[j]],
                         rows_v.at[pl.ds(j * CHUNK, CHUNK)], sem)
        return carry

    lax.fori_loop(0, NGC, fire, 0)

    def drain(j, carry):
        pltpu.make_async_copy(cb_hbm.at[g_v.at[j]],
                              rows_v.at[pl.ds(j * CHUNK, CHUNK)], sem).wait()
        return carry

    lax.fori_loop(0, NGC, drain, 0)

    pltpu.sync_copy(rows_v, out_hbm.at[pl.ds(wid * BPW * NCB, BPW * NCB)])


_pq_decode = functools.partial(
    pl.kernel,
    out_type=jax.ShapeDtypeStruct((BATCH * NCB, SUB), jnp.float32),
    mesh=plsc.VectorSubcoreMesh(core_axis_name="c", subcore_axis_name="s"),
    compiler_params=pltpu.CompilerParams(needs_layout_passes=False,
                                         use_tc_tiling_on_sc=False),
    scratch_types=[
        pltpu.VMEM((NIC, CHUNK), jnp.int32),
        pltpu.VMEM((BPW, NCB), jnp.int32),
        pltpu.VMEM((NGC, CHUNK), jnp.int32),
        pltpu.VMEM((BPW * NCB, SUB), jnp.float32),
        pltpu.SemaphoreType.DMA,
    ],
)(_pq_body)


def kernel(indices, codebooks, codes):
    idx2 = indices.astype(jnp.int32).reshape(BATCH // CHUNK, CHUNK)
    cb_flat = codebooks.reshape(NCB * CBS, SUB)
    out = _pq_decode(idx2, cb_flat, codes)
    return out.reshape(BATCH, DIM)
